# Initial kernel scaffold; baseline (speedup 1.0000x reference)
import jax
import jax.numpy as jnp
from jax.experimental import pallas as pl


def kernel(X, params, graph_segment_ids, W_in, b_in, W_pred, b_pred):
    raise NotImplementedError("write your pallas kernel here")


# speedup vs baseline: 2.7479x; 2.7479x over previous
import functools

import jax
import jax.numpy as jnp
from jax import lax
from jax.experimental import pallas as pl
from jax.experimental.pallas import tpu as pltpu
from jax.experimental.pallas import tpu_sc as plsc

N_NODES = 50000
D_IN = 128
HIDDEN = 256
OUT = 128
NUM_GRAPHS = 256

NC = 2
NS = 16
NW = NC * NS
BLK = 80
NBLK = N_NODES // BLK
BASE_BLK = NBLK // NW
EXTRA = NBLK - BASE_BLK * NW
CW = D_IN + 16
ACC_LEN = NUM_GRAPHS * CW
RED = 16 * CW


@functools.partial(
    pl.kernel,
    out_type=jax.ShapeDtypeStruct((NC, ACC_LEN), jnp.float32),
    mesh=plsc.VectorSubcoreMesh(core_axis_name="c", subcore_axis_name="s"),
    scratch_types=[
        pltpu.VMEM((BLK * D_IN,), jnp.float32),
        pltpu.VMEM((BLK + 16,), jnp.int32),
        pltpu.VMEM((ACC_LEN,), jnp.float32),
        pltpu.VMEM_SHARED((NS, ACC_LEN), jnp.float32),
        pltpu.VMEM((RED,), jnp.float32),
        pltpu.VMEM((RED,), jnp.float32),
    ],
)
def _sc_segsum(x_hbm, ids_hbm, out_hbm, xbuf, idbuf, acc, shared, rsum, rtmp):
    c = lax.axis_index("c")
    s = lax.axis_index("s")
    w = c * NS + s

    iota = lax.iota(jnp.int32, 16)
    cntv = jnp.where(iota == 0, 1.0, 0.0).astype(jnp.float32)
    zeros16 = jnp.zeros((16,), jnp.float32)

    def zero_body(i, carry):
        acc[pl.ds(i * 16, 16)] = zeros16
        return carry

    lax.fori_loop(0, ACC_LEN // 16, zero_body, 0)

    start = w * BASE_BLK + jnp.minimum(w, EXTRA)
    nblk = jnp.where(w < EXTRA, BASE_BLK + 1, BASE_BLK)

    def blk_body(i, carry):
        blk = start + i
        pltpu.sync_copy(x_hbm.at[pl.ds(blk * (BLK * D_IN), BLK * D_IN)], xbuf)
        pltpu.sync_copy(ids_hbm.at[pl.ds(blk * BLK, BLK)], idbuf.at[pl.ds(0, BLK)])

        def row_body(r, rc):
            seg = idbuf[pl.ds(r, 16)][0]
            base = seg * CW
            for cg in range(D_IN // 16):
                v = xbuf[pl.ds(r * D_IN + cg * 16, 16)]
                acc[pl.ds(base + cg * 16, 16)] = acc[pl.ds(base + cg * 16, 16)] + v
            acc[pl.ds(base + D_IN, 16)] = acc[pl.ds(base + D_IN, 16)] + cntv
            return rc

        lax.fori_loop(0, BLK, row_body, 0)
        return carry

    lax.fori_loop(0, nblk, blk_body, 0)

    pltpu.sync_copy(acc, shared.at[s])
    plsc.subcore_barrier()

    pltpu.sync_copy(shared.at[0, pl.ds(s * RED, RED)], rsum)
    for p in range(1, NS):
        pltpu.sync_copy(shared.at[p, pl.ds(s * RED, RED)], rtmp)

        def add_body(i, carry):
            j = i * 16
            rsum[pl.ds(j, 16)] = rsum[pl.ds(j, 16)] + rtmp[pl.ds(j, 16)]
            return carry

        lax.fori_loop(0, RED // 16, add_body, 0)

    pltpu.sync_copy(rsum, out_hbm.at[c, pl.ds(s * RED, RED)])


def _tc_body(sacc_ref, w_in_ref, b16_ref, w_pred_ref, b_pred_ref, out_ref):
    a = sacc_ref[0] + sacc_ref[1]
    sx = a[:, :D_IN]
    ext = a[:, D_IN:]
    pooled = jnp.dot(sx, w_in_ref[...], preferred_element_type=jnp.float32)
    pooled = pooled + jnp.dot(ext, b16_ref[...],
                              preferred_element_type=jnp.float32)
    out_ref[...] = (
        jnp.dot(pooled, w_pred_ref[...], preferred_element_type=jnp.float32)
        + b_pred_ref[...]
    )


def kernel(X, params, graph_segment_ids, W_in, b_in, W_pred, b_pred):
    del params
    x_flat = X.reshape(-1)
    ids32 = graph_segment_ids.astype(jnp.int32)
    sacc = _sc_segsum(x_flat, ids32).reshape(NC, NUM_GRAPHS, CW)
    b16 = jnp.zeros((16, HIDDEN), jnp.float32).at[0].set(b_in)
    out = pl.pallas_call(
        _tc_body,
        out_shape=jax.ShapeDtypeStruct((NUM_GRAPHS, OUT), jnp.float32),
    )(sacc, W_in, b16, W_pred, b_pred.reshape(1, OUT))
    return out


# speedup vs baseline: 3.5951x; 1.3083x over previous
import functools

import jax
import jax.numpy as jnp
from jax import lax
from jax.experimental import pallas as pl
from jax.experimental.pallas import tpu as pltpu
from jax.experimental.pallas import tpu_sc as plsc

N_NODES = 50000
D_IN = 128
HIDDEN = 256
OUT = 128
NUM_GRAPHS = 256

NC = 2
NS = 16
NW = NC * NS
BLK = 80
NBLK = N_NODES // BLK
BASE_BLK = NBLK // NW
EXTRA = NBLK - BASE_BLK * NW
CW = D_IN + 16
ACC_LEN = NUM_GRAPHS * CW
RED = 16 * CW


@functools.partial(
    pl.kernel,
    out_type=jax.ShapeDtypeStruct((NC, ACC_LEN), jnp.float32),
    mesh=plsc.VectorSubcoreMesh(core_axis_name="c", subcore_axis_name="s"),
    scratch_types=[
        pltpu.VMEM((2, BLK * D_IN), jnp.float32),
        pltpu.VMEM((2, BLK + 16), jnp.int32),
        pltpu.VMEM((ACC_LEN,), jnp.float32),
        pltpu.VMEM_SHARED((NS, ACC_LEN), jnp.float32),
        pltpu.VMEM((RED,), jnp.float32),
        pltpu.VMEM((NS // 2, RED), jnp.float32),
        pltpu.SemaphoreType.DMA,
        pltpu.SemaphoreType.DMA,
    ],
)
def _sc_segsum(x_hbm, ids_hbm, out_hbm, xbuf, idbuf, acc, shared, rsum, rtmp,
               semx, semi):
    c = lax.axis_index("c")
    s = lax.axis_index("s")
    w = c * NS + s

    iota = lax.iota(jnp.int32, 16)
    cntv = jnp.where(iota == 0, 1.0, 0.0).astype(jnp.float32)
    zeros16 = jnp.zeros((16,), jnp.float32)

    def zero_body(i, carry):
        acc[pl.ds(i * 16, 16)] = zeros16
        return carry

    lax.fori_loop(0, ACC_LEN // 16, zero_body, 0, unroll=8)

    start = w * BASE_BLK + jnp.minimum(w, EXTRA)
    nblk = jnp.where(w < EXTRA, BASE_BLK + 1, BASE_BLK)

    def issue(i, par):
        blk = start + i
        pltpu.async_copy(
            x_hbm.at[pl.ds(blk * (BLK * D_IN), BLK * D_IN)], xbuf.at[par], semx)
        pltpu.async_copy(
            ids_hbm.at[pl.ds(blk * BLK, BLK)], idbuf.at[par, pl.ds(0, BLK)],
            semi)

    def drain(i, par):
        blk = start + i
        pltpu.make_async_copy(
            x_hbm.at[pl.ds(blk * (BLK * D_IN), BLK * D_IN)], xbuf.at[par],
            semx).wait()
        pltpu.make_async_copy(
            ids_hbm.at[pl.ds(blk * BLK, BLK)], idbuf.at[par, pl.ds(0, BLK)],
            semi).wait()

    issue(0, 0)

    def blk_body(i, carry):
        par = jnp.bitwise_and(i, 1)
        drain(i, par)

        @pl.when(i + 1 < nblk)
        def _():
            issue(i + 1, jnp.bitwise_xor(par, 1))

        def row_body(r, rc):
            seg = idbuf[par, pl.ds(r, 16)][0]
            base = seg * CW
            for cg in range(D_IN // 16):
                v = xbuf[par, pl.ds(r * D_IN + cg * 16, 16)]
                acc[pl.ds(base + cg * 16, 16)] = (
                    acc[pl.ds(base + cg * 16, 16)] + v)
            acc[pl.ds(base + D_IN, 16)] = acc[pl.ds(base + D_IN, 16)] + cntv
            return rc

        lax.fori_loop(0, BLK, row_body, 0, unroll=4)
        return carry

    lax.fori_loop(0, nblk, blk_body, 0)

    pltpu.sync_copy(acc, shared.at[s])
    plsc.subcore_barrier()

    pltpu.sync_copy(shared.at[pl.ds(0, NS // 2), pl.ds(s * RED, RED)], rtmp)

    def add_body0(i, carry):
        j = i * 16
        v = rtmp[0, pl.ds(j, 16)]
        for p in range(1, NS // 2):
            v = v + rtmp[p, pl.ds(j, 16)]
        rsum[pl.ds(j, 16)] = v
        return carry

    lax.fori_loop(0, RED // 16, add_body0, 0, unroll=2)

    pltpu.sync_copy(shared.at[pl.ds(NS // 2, NS // 2), pl.ds(s * RED, RED)],
                    rtmp)

    def add_body1(i, carry):
        j = i * 16
        v = rtmp[0, pl.ds(j, 16)]
        for p in range(1, NS // 2):
            v = v + rtmp[p, pl.ds(j, 16)]
        rsum[pl.ds(j, 16)] = rsum[pl.ds(j, 16)] + v
        return carry

    lax.fori_loop(0, RED // 16, add_body1, 0, unroll=2)

    pltpu.sync_copy(rsum, out_hbm.at[c, pl.ds(s * RED, RED)])


def _tc_body(sacc_ref, w_in_ref, b16_ref, w_pred_ref, b_pred_ref, out_ref):
    a = sacc_ref[0] + sacc_ref[1]
    sx = a[:, :D_IN]
    ext = a[:, D_IN:]
    pooled = jnp.dot(sx, w_in_ref[...], preferred_element_type=jnp.float32)
    pooled = pooled + jnp.dot(ext, b16_ref[...],
                              preferred_element_type=jnp.float32)
    out_ref[...] = (
        jnp.dot(pooled, w_pred_ref[...], preferred_element_type=jnp.float32)
        + b_pred_ref[...]
    )


def kernel(X, params, graph_segment_ids, W_in, b_in, W_pred, b_pred):
    del params
    x_flat = X.reshape(-1)
    ids32 = graph_segment_ids.astype(jnp.int32)
    sacc = _sc_segsum(x_flat, ids32).reshape(NC, NUM_GRAPHS, CW)
    b16 = jnp.zeros((16, HIDDEN), jnp.float32).at[0].set(b_in)
    out = pl.pallas_call(
        _tc_body,
        out_shape=jax.ShapeDtypeStruct((NUM_GRAPHS, OUT), jnp.float32),
    )(sacc, W_in, b16, W_pred, b_pred.reshape(1, OUT))
    return out
